# baseline (speedup 1.0000x reference)
import functools

import jax
import jax.numpy as jnp
from jax import lax
from jax.experimental import pallas as pl
from jax.experimental.pallas import tpu as pltpu
from jax.experimental.pallas import tpu_sc as plsc

MAX_OBJ = 100
THRESH = 0.1
B, H, W, C = 16, 128, 128, 80
WC = W * C
NELEM = H * WC
NWORDS = (H // 8) * (WC // 4)
CAP = 256
NPAD = 112
LANES = 16


def _tc_body(x_ref, w_ref):
    x = x_ref[0]
    zw = jnp.zeros((H, C, 1), x.dtype)
    m = jnp.maximum(x, jnp.maximum(
        jnp.concatenate([x[:, :, 1:], zw], axis=2),
        jnp.concatenate([zw, x[:, :, : W - 1]], axis=2)))
    zh = jnp.zeros((1, C, W), x.dtype)
    hmax = jnp.maximum(m, jnp.maximum(
        jnp.concatenate([m[1:], zh], axis=0),
        jnp.concatenate([zh, m[: H - 1]], axis=0)))
    nms = jnp.where(hmax == x, x, 0.0)

    gm = jnp.max(nms, axis=2)
    gmi = lax.bitcast_convert_type(gm, jnp.int32)

    def bs_body(_, carry):
        lo, hi = carry
        mid = lo + (hi - lo) // 2
        cnt = jnp.sum((gmi >= mid).astype(jnp.int32))
        big = cnt >= MAX_OBJ
        return (jnp.where(big, mid, lo), jnp.where(big, hi, mid))

    lo, _ = lax.fori_loop(0, 31, bs_body, (jnp.int32(0), jnp.int32(0x7F800000)))

    nmsi = lax.bitcast_convert_type(nms, jnp.int32)
    mask = ((nmsi >= lo) & (nms > THRESH)).astype(jnp.int32)

    wts = (1 << lax.broadcasted_iota(jnp.int32, (1, 1, 16), 2))
    g16 = [jnp.sum(mask[:, :, 16 * j: 16 * j + 16] * wts, axis=2)
           for j in range(8)]
    wrds = [g16[2 * j] | (g16[2 * j + 1] << 16) for j in range(4)]
    w_ref[0] = jnp.stack(wrds, axis=0)


def _lex_best(cval_ref, cidx_ref, t, carry):
    bv, bi, bp = carry
    sl = pl.ds(t * LANES, LANES)
    v = cval_ref[sl]
    ci = cidx_ref[sl]
    pos = lax.iota(jnp.int32, LANES) + t * LANES
    better = (v > bv) | ((v == bv) & (ci < bi))
    return (jnp.where(better, v, bv), jnp.where(better, ci, bi),
            jnp.where(better, pos, bp))


def _sc_body(words_hbm, heat_hbm, offs_hbm, sizes_hbm, out_hbm,
             words_v, cidx_v, gidx_v, cval_v, resv_v, resi_v, sp0_v, sp1_v,
             offy_v, offx_v, sizy_v, sizx_v, outbuf_v, sem):
    wid = lax.axis_index("s") * 2 + lax.axis_index("c")
    lane = lax.iota(jnp.int32, LANES)
    zeros16 = jnp.zeros((LANES,), jnp.int32)

    @pl.when(wid < B)
    def _():
        b = wid
        pltpu.sync_copy(words_hbm.at[b], words_v)
        for t in range(CAP // LANES):
            cidx_v[pl.ds(t * LANES, LANES)] = zeros16

        def extract_words(i, rem, cursor):
            def cond(c):
                rem, _ = c
                return jnp.any(rem != 0)

            def body(c):
                rem, cur = c
                jv = plsc.all_reduce_ffs(rem != 0)
                w = plsc.load_gather(words_v, [i * LANES + jv])
                p16 = i * LANES + jv
                g = p16 // (H * C)
                rem2 = p16 % (H * C)
                y = rem2 // C
                c = rem2 % C
                e0 = y * WC + (32 * g + lane) * C + c
                e1 = e0 + LANES * C
                m0 = ((w >> lane) & 1) == 1
                m1 = ((w >> (lane + 16)) & 1) == 1
                p0 = cur + plsc.cumsum(m0.astype(jnp.int32)) - 1
                n0 = jnp.max(plsc.all_reduce_population_count(m0))
                p1 = cur + n0 + plsc.cumsum(m1.astype(jnp.int32)) - 1
                n1 = jnp.max(plsc.all_reduce_population_count(m1))
                plsc.store_scatter(cidx_v, [p0], e0, mask=m0 & (p0 < CAP))
                plsc.store_scatter(cidx_v, [p1], e1, mask=m1 & (p1 < CAP))
                rem = jnp.where(lane == jv, 0, rem)
                return rem, cur + n0 + n1

            _, cursor = lax.while_loop(cond, body, (rem, cursor))
            return cursor

        def scan_step(i, cursor):
            w16 = words_v[pl.ds(i * LANES, LANES)]
            return lax.cond(
                jnp.any(w16 != 0),
                lambda c: extract_words(i, w16, c),
                lambda c: c, cursor)

        ncand = lax.fori_loop(0, NWORDS // LANES, scan_step, jnp.int32(0))
        ncand = jnp.minimum(ncand, CAP)

        for t in range(CAP // LANES):
            sl = pl.ds(t * LANES, LANES)
            e = cidx_v[sl]
            y = e // WC
            r = e % WC
            xx = r // C
            cc = r % C
            gidx_v[sl] = ((b * H + y) * C + cc) * W + xx
        pltpu.async_copy(heat_hbm.at[gidx_v], cval_v, sem).wait()
        for t in range(CAP // LANES):
            sl = pl.ds(t * LANES, LANES)
            pos = lane + t * LANES
            cval_v[sl] = jnp.where(pos < ncand, cval_v[sl], -1.0)

        INTMAX = jnp.int32(2147483647)

        def select_step(k, _):
            bv = jnp.full((LANES,), -2.0, jnp.float32)
            bi = jnp.full((LANES,), INTMAX)
            bp = jnp.full((LANES,), INTMAX)
            for t in range(CAP // LANES):
                bv, bi, bp = _lex_best(cval_v, cidx_v, t, (bv, bi, bp))
            mval = jnp.max(bv)
            vm = bv == mval
            istar = jnp.min(jnp.where(vm, bi, INTMAX))
            pstar = jnp.min(jnp.where(vm & (bi == istar), bp, INTMAX))
            pvec = jnp.full((LANES,), pstar)
            km = lane == 0
            kvec = jnp.full((LANES,), k)
            plsc.store_scatter(resi_v, [kvec], jnp.full((LANES,), istar),
                               mask=km)
            plsc.store_scatter(resv_v, [kvec], jnp.full((LANES,), mval), mask=km)
            plsc.store_scatter(cval_v, [pvec],
                               jnp.full((LANES,), -2.0), mask=km)
            return 0

        lax.fori_loop(0, MAX_OBJ, select_step, 0)

        for t in range(NPAD // LANES):
            sl = pl.ds(t * LANES, LANES)
            kpos = lane + t * LANES
            idx = resi_v[sl]
            sp = jnp.where(kpos < MAX_OBJ, idx // C, 0)
            y = sp // W
            xx = sp % W
            g = ((b * H + y) * 2) * W + xx
            sp0_v[sl] = g
            sp1_v[sl] = g + W
        pltpu.async_copy(offs_hbm.at[sp0_v], offy_v, sem).wait()
        pltpu.async_copy(offs_hbm.at[sp1_v], offx_v, sem).wait()
        pltpu.async_copy(sizes_hbm.at[sp0_v], sizy_v, sem).wait()
        pltpu.async_copy(sizes_hbm.at[sp1_v], sizx_v, sem).wait()

        for t in range(NPAD // LANES):
            sl = pl.ds(t * LANES, LANES)
            kpos = lane + t * LANES
            idx = resi_v[sl]
            sc = resv_v[sl]
            cls = (idx % C).astype(jnp.float32)
            sp = idx // C
            xs = (sp % W).astype(jnp.float32)
            ys = (sp // W).astype(jnp.float32)
            oy = offy_v[sl]
            ox = offx_v[sl]
            sy = sizy_v[sl]
            sx = sizx_v[sl]
            cy = ys + oy
            cx = xs + ox
            hf = jnp.float32(H)
            wf = jnp.float32(W)
            y1 = jnp.clip(cy - sy * 0.5, 0.0, hf) * (1.0 / hf)
            y2 = jnp.clip(cy + sy * 0.5, 0.0, hf) * (1.0 / hf)
            x1 = jnp.clip(cx - sx * 0.5, 0.0, wf) * (1.0 / wf)
            x2 = jnp.clip(cx + sx * 0.5, 0.0, wf) * (1.0 / wf)
            keep = sc > THRESH
            okm = kpos < MAX_OBJ
            base6 = kpos * 6
            for col, val in enumerate((y1, x1, y2, x2, sc, cls)):
                plsc.store_scatter(outbuf_v, [base6 + col],
                                   jnp.where(keep, val, 0.0), mask=okm)
        pltpu.sync_copy(outbuf_v, out_hbm.at[b])


def kernel(heatmaps, boxsizes, offsets):
    hmT = jnp.transpose(heatmaps, (0, 1, 3, 2))
    words = pl.pallas_call(
        _tc_body,
        grid=(B,),
        in_specs=[pl.BlockSpec((1, H, C, W), lambda i: (i, 0, 0, 0))],
        out_specs=pl.BlockSpec((1, 4, H, C), lambda i: (i, 0, 0, 0)),
        out_shape=jax.ShapeDtypeStruct((B, 4, H, C), jnp.int32),
    )(hmT)

    mesh = plsc.VectorSubcoreMesh(core_axis_name="c", subcore_axis_name="s")
    sc = functools.partial(
        pl.kernel, mesh=mesh,
        compiler_params=pltpu.CompilerParams(needs_layout_passes=False),
        out_type=jax.ShapeDtypeStruct((B, 640), jnp.float32),
        scratch_types=[
            pltpu.VMEM((NWORDS,), jnp.int32),
            pltpu.VMEM((CAP,), jnp.int32),
            pltpu.VMEM((CAP,), jnp.int32),
            pltpu.VMEM((CAP,), jnp.float32),
            pltpu.VMEM((NPAD,), jnp.float32),
            pltpu.VMEM((NPAD,), jnp.int32),
            pltpu.VMEM((NPAD,), jnp.int32),
            pltpu.VMEM((NPAD,), jnp.int32),
            pltpu.VMEM((NPAD,), jnp.float32),
            pltpu.VMEM((NPAD,), jnp.float32),
            pltpu.VMEM((NPAD,), jnp.float32),
            pltpu.VMEM((NPAD,), jnp.float32),
            pltpu.VMEM((640,), jnp.float32),
            pltpu.SemaphoreType.DMA,
        ],
    )(_sc_body)
    out = sc(
        words.reshape(B, NWORDS),
        hmT.reshape(B * NELEM),
        jnp.transpose(offsets, (0, 1, 3, 2)).reshape(B * H * W * 2),
        jnp.transpose(boxsizes, (0, 1, 3, 2)).reshape(B * H * W * 2),
    )
    return out[:, : MAX_OBJ * 6].reshape(B, MAX_OBJ, 6)


# speedup vs baseline: 2.3055x; 2.3055x over previous
import functools

import jax
import jax.numpy as jnp
from jax import lax
from jax.experimental import pallas as pl
from jax.experimental.pallas import tpu as pltpu
from jax.experimental.pallas import tpu_sc as plsc

MAX_OBJ = 100
THRESH = 0.1
B, H, W, C = 16, 128, 128, 80
WC = W * C
NELEM = H * WC
NWORDS = (H // 8) * (WC // 4)
CAP = 256
NPAD = 112
LANES = 16


def _tc_body(x_ref, w_ref):
    x = x_ref[0]
    zw = jnp.zeros((H, 1, C), x.dtype)
    m = jnp.maximum(x, jnp.maximum(
        jnp.concatenate([x[:, 1:], zw], axis=1),
        jnp.concatenate([zw, x[:, : W - 1]], axis=1)))
    zh = jnp.zeros((1, W, C), x.dtype)
    hmax = jnp.maximum(m, jnp.maximum(
        jnp.concatenate([m[1:], zh], axis=0),
        jnp.concatenate([zh, m[: H - 1]], axis=0)))
    nms = jnp.where(hmax == x, x, 0.0)

    gm = jnp.max(nms, axis=1)
    gmi = lax.bitcast_convert_type(gm, jnp.int32)

    def bs_body(_, carry):
        lo, hi = carry
        mid = lo + (hi - lo) // 2
        cnt = jnp.sum((gmi >= mid).astype(jnp.int32))
        big = cnt >= MAX_OBJ
        return (jnp.where(big, mid, lo), jnp.where(big, hi, mid))

    lo, _ = lax.fori_loop(0, 31, bs_body, (jnp.int32(0), jnp.int32(0x7F800000)))

    nmsi = lax.bitcast_convert_type(nms, jnp.int32)
    mask = ((nmsi >= lo) & (nms > THRESH)).astype(jnp.int32)

    wts = (1 << lax.broadcasted_iota(jnp.int32, (1, 16, 1), 1))
    g16 = [jnp.sum(mask[:, 16 * j: 16 * j + 16, :] * wts, axis=1)
           for j in range(8)]
    wrds = [g16[2 * j] | (g16[2 * j + 1] << 16) for j in range(4)]
    w_ref[0] = jnp.stack(wrds, axis=0)


def _lex_best(cval_ref, cidx_ref, t, carry):
    bv, bi, bp = carry
    sl = pl.ds(t * LANES, LANES)
    v = cval_ref[sl]
    ci = cidx_ref[sl]
    pos = lax.iota(jnp.int32, LANES) + t * LANES
    better = (v > bv) | ((v == bv) & (ci < bi))
    return (jnp.where(better, v, bv), jnp.where(better, ci, bi),
            jnp.where(better, pos, bp))


def _sc_body(words_hbm, heat_hbm, offs_hbm, sizes_hbm, out_hbm,
             words_v, cidx_v, gidx_v, cval_v, resv_v, resi_v, sp0_v, sp1_v,
             offy_v, offx_v, sizy_v, sizx_v, outbuf_v, sem):
    wid = lax.axis_index("s") * 2 + lax.axis_index("c")
    lane = lax.iota(jnp.int32, LANES)
    zeros16 = jnp.zeros((LANES,), jnp.int32)

    @pl.when(wid < B)
    def _():
        b = wid
        pltpu.sync_copy(words_hbm.at[b], words_v)
        for t in range(CAP // LANES):
            cidx_v[pl.ds(t * LANES, LANES)] = zeros16

        def extract_words(i, rem, cursor):
            def cond(c):
                rem, _ = c
                return jnp.any(rem != 0)

            def body(c):
                rem, cur = c
                jv = plsc.all_reduce_ffs(rem != 0)
                w = plsc.load_gather(words_v, [i * LANES + jv])
                p16 = i * LANES + jv
                g = p16 // (H * C)
                rem2 = p16 % (H * C)
                y = rem2 // C
                c = rem2 % C
                e0 = y * WC + (32 * g + lane) * C + c
                e1 = e0 + LANES * C
                m0 = ((w >> lane) & 1) == 1
                m1 = ((w >> (lane + 16)) & 1) == 1
                p0 = cur + plsc.cumsum(m0.astype(jnp.int32)) - 1
                n0 = jnp.max(plsc.all_reduce_population_count(m0))
                p1 = cur + n0 + plsc.cumsum(m1.astype(jnp.int32)) - 1
                n1 = jnp.max(plsc.all_reduce_population_count(m1))
                plsc.store_scatter(cidx_v, [p0], e0, mask=m0 & (p0 < CAP))
                plsc.store_scatter(cidx_v, [p1], e1, mask=m1 & (p1 < CAP))
                rem = jnp.where(lane == jv, 0, rem)
                return rem, cur + n0 + n1

            _, cursor = lax.while_loop(cond, body, (rem, cursor))
            return cursor

        def scan_step(i, cursor):
            w16 = words_v[pl.ds(i * LANES, LANES)]
            return lax.cond(
                jnp.any(w16 != 0),
                lambda c: extract_words(i, w16, c),
                lambda c: c, cursor)

        ncand = lax.fori_loop(0, NWORDS // LANES, scan_step, jnp.int32(0))
        ncand = jnp.minimum(ncand, CAP)

        for t in range(CAP // LANES):
            sl = pl.ds(t * LANES, LANES)
            e = cidx_v[sl]
            y = e // WC
            r = e % WC
            xx = r // C
            cc = r % C
            gidx_v[sl] = ((b * H + y) * C + cc) * W + xx
        pltpu.async_copy(heat_hbm.at[gidx_v], cval_v, sem).wait()
        for t in range(CAP // LANES):
            sl = pl.ds(t * LANES, LANES)
            pos = lane + t * LANES
            cval_v[sl] = jnp.where(pos < ncand, cval_v[sl], -1.0)

        INTMAX = jnp.int32(2147483647)

        def select_step(k, _):
            bv = jnp.full((LANES,), -2.0, jnp.float32)
            bi = jnp.full((LANES,), INTMAX)
            bp = jnp.full((LANES,), INTMAX)
            for t in range(CAP // LANES):
                bv, bi, bp = _lex_best(cval_v, cidx_v, t, (bv, bi, bp))
            mval = jnp.max(bv)
            vm = bv == mval
            istar = jnp.min(jnp.where(vm, bi, INTMAX))
            pstar = jnp.min(jnp.where(vm & (bi == istar), bp, INTMAX))
            pvec = jnp.full((LANES,), pstar)
            km = lane == 0
            kvec = jnp.full((LANES,), k)
            plsc.store_scatter(resi_v, [kvec], jnp.full((LANES,), istar),
                               mask=km)
            plsc.store_scatter(resv_v, [kvec], jnp.full((LANES,), mval), mask=km)
            plsc.store_scatter(cval_v, [pvec],
                               jnp.full((LANES,), -2.0), mask=km)
            return 0

        lax.fori_loop(0, MAX_OBJ, select_step, 0)

        for t in range(NPAD // LANES):
            sl = pl.ds(t * LANES, LANES)
            kpos = lane + t * LANES
            idx = resi_v[sl]
            sp = jnp.where(kpos < MAX_OBJ, idx // C, 0)
            y = sp // W
            xx = sp % W
            g = ((b * H + y) * 2) * W + xx
            sp0_v[sl] = g
            sp1_v[sl] = g + W
        pltpu.async_copy(offs_hbm.at[sp0_v], offy_v, sem).wait()
        pltpu.async_copy(offs_hbm.at[sp1_v], offx_v, sem).wait()
        pltpu.async_copy(sizes_hbm.at[sp0_v], sizy_v, sem).wait()
        pltpu.async_copy(sizes_hbm.at[sp1_v], sizx_v, sem).wait()

        for t in range(NPAD // LANES):
            sl = pl.ds(t * LANES, LANES)
            kpos = lane + t * LANES
            idx = resi_v[sl]
            sc = resv_v[sl]
            cls = (idx % C).astype(jnp.float32)
            sp = idx // C
            xs = (sp % W).astype(jnp.float32)
            ys = (sp // W).astype(jnp.float32)
            oy = offy_v[sl]
            ox = offx_v[sl]
            sy = sizy_v[sl]
            sx = sizx_v[sl]
            cy = ys + oy
            cx = xs + ox
            hf = jnp.float32(H)
            wf = jnp.float32(W)
            y1 = jnp.clip(cy - sy * 0.5, 0.0, hf) * (1.0 / hf)
            y2 = jnp.clip(cy + sy * 0.5, 0.0, hf) * (1.0 / hf)
            x1 = jnp.clip(cx - sx * 0.5, 0.0, wf) * (1.0 / wf)
            x2 = jnp.clip(cx + sx * 0.5, 0.0, wf) * (1.0 / wf)
            keep = sc > THRESH
            okm = kpos < MAX_OBJ
            base6 = kpos * 6
            for col, val in enumerate((y1, x1, y2, x2, sc, cls)):
                plsc.store_scatter(outbuf_v, [base6 + col],
                                   jnp.where(keep, val, 0.0), mask=okm)
        pltpu.sync_copy(outbuf_v, out_hbm.at[b])


def kernel(heatmaps, boxsizes, offsets):
    hmT = jnp.transpose(heatmaps, (0, 1, 3, 2))
    words = pl.pallas_call(
        _tc_body,
        grid=(B,),
        in_specs=[pl.BlockSpec((1, H, W, C), lambda i: (i, 0, 0, 0))],
        out_specs=pl.BlockSpec((1, 4, H, C), lambda i: (i, 0, 0, 0)),
        out_shape=jax.ShapeDtypeStruct((B, 4, H, C), jnp.int32),
    )(heatmaps)

    mesh = plsc.VectorSubcoreMesh(core_axis_name="c", subcore_axis_name="s")
    sc = functools.partial(
        pl.kernel, mesh=mesh,
        compiler_params=pltpu.CompilerParams(needs_layout_passes=False),
        out_type=jax.ShapeDtypeStruct((B, 640), jnp.float32),
        scratch_types=[
            pltpu.VMEM((NWORDS,), jnp.int32),
            pltpu.VMEM((CAP,), jnp.int32),
            pltpu.VMEM((CAP,), jnp.int32),
            pltpu.VMEM((CAP,), jnp.float32),
            pltpu.VMEM((NPAD,), jnp.float32),
            pltpu.VMEM((NPAD,), jnp.int32),
            pltpu.VMEM((NPAD,), jnp.int32),
            pltpu.VMEM((NPAD,), jnp.int32),
            pltpu.VMEM((NPAD,), jnp.float32),
            pltpu.VMEM((NPAD,), jnp.float32),
            pltpu.VMEM((NPAD,), jnp.float32),
            pltpu.VMEM((NPAD,), jnp.float32),
            pltpu.VMEM((640,), jnp.float32),
            pltpu.SemaphoreType.DMA,
        ],
    )(_sc_body)
    out = sc(
        words.reshape(B, NWORDS),
        hmT.reshape(B * NELEM),
        jnp.transpose(offsets, (0, 1, 3, 2)).reshape(B * H * W * 2),
        jnp.transpose(boxsizes, (0, 1, 3, 2)).reshape(B * H * W * 2),
    )
    return out[:, : MAX_OBJ * 6].reshape(B, MAX_OBJ, 6)
